# Initial kernel scaffold; baseline (speedup 1.0000x reference)
import jax
import jax.numpy as jnp
from jax.experimental import pallas as pl


def kernel(x, emb_weight):
    raise NotImplementedError("write your pallas kernel here")


# speedup vs baseline: 6.0681x; 6.0681x over previous
import functools

import jax
import jax.numpy as jnp
from jax import lax
from jax.experimental import pallas as pl
from jax.experimental.pallas import tpu as pltpu
from jax.experimental.pallas import tpu_sc as plsc

_NUM_EMB = 8192
_DIM = 256
_COMMIT = 0.25
_BM = 512
_BN = 2048


def _vq_dist_argmin_body(total_elems, x_ref, e_ref, idx_ref, loss_ref,
                         minval, minidx, lossval, acc):
    c = pl.program_id(0)
    r = pl.program_id(1)
    nc = pl.num_programs(0)
    nr = pl.num_programs(1)
    rows = pl.ds(r * _BM, _BM)

    @pl.when(jnp.logical_and(c == 0, r == 0))
    def _():
        acc[0] = jnp.float32(0.0)

    @pl.when(c == 0)
    def _():
        minval[rows, :] = jnp.full((_BM, 1), jnp.inf, jnp.float32)
        minidx[rows, :] = jnp.zeros((_BM, 1), jnp.int32)
        lossval[rows, :] = jnp.zeros((_BM, 1), jnp.float32)

    xb = x_ref[...]
    eb = e_ref[...]
    x2 = jnp.sum(xb * xb, axis=1, keepdims=True)
    e2 = jnp.sum(eb * eb, axis=1)[None, :]
    ab = lax.dot_general(xb.astype(jnp.bfloat16), eb.astype(jnp.bfloat16),
                         (((1,), (1,)), ((), ())),
                         preferred_element_type=jnp.float32)
    d2 = (x2 + e2) - 2.0 * ab
    dist = jnp.sqrt(jnp.maximum(d2, 0.0))

    lmin = jnp.min(dist, axis=1, keepdims=True)
    cols = lax.broadcasted_iota(jnp.int32, dist.shape, 1)
    larg = jnp.min(jnp.where(dist == lmin, cols, _NUM_EMB), axis=1,
                   keepdims=True) + c * _BN

    prev = minval[rows, :]
    better = lmin < prev
    minidx[rows, :] = jnp.where(better, larg, minidx[rows, :])
    minval[rows, :] = jnp.where(better, lmin, prev).astype(jnp.bfloat16).astype(jnp.float32)
    lossval[rows, :] = jnp.where(better, lmin, lossval[rows, :])

    @pl.when(c == nc - 1)
    def _():
        idx_ref[...] = minidx[rows, :]
        lv = lossval[rows, :]
        acc[0] = acc[0] + jnp.sum(lv * lv)

        @pl.when(r == nr - 1)
        def _():
            loss_ref[0] = (acc[0] * jnp.float32(1.0 + _COMMIT)
                           / jnp.float32(total_elems))


def _dist_argmin(flat, emb_weight):
    n = flat.shape[0]
    grid = (_NUM_EMB // _BN, n // _BM)
    return pl.pallas_call(
        functools.partial(_vq_dist_argmin_body, n * _DIM),
        grid=grid,
        in_specs=[
            pl.BlockSpec((_BM, _DIM), lambda c, r: (r, 0)),
            pl.BlockSpec((_BN, _DIM), lambda c, r: (c, 0)),
        ],
        out_specs=[
            pl.BlockSpec((_BM, 1), lambda c, r: (r, 0)),
            pl.BlockSpec(memory_space=pltpu.SMEM),
        ],
        out_shape=[
            jax.ShapeDtypeStruct((n, 1), jnp.int32),
            jax.ShapeDtypeStruct((1,), jnp.float32),
        ],
        scratch_shapes=[
            pltpu.VMEM((n, 1), jnp.float32),
            pltpu.VMEM((n, 1), jnp.int32),
            pltpu.VMEM((n, 1), jnp.float32),
            pltpu.SMEM((1,), jnp.float32),
        ],
    )(flat, emb_weight)


def _sc_gather(idx_flat, table):
    n = idx_flat.shape[0]
    d = table.shape[1]
    info = plsc.get_sparse_core_info()
    num_cores = info.num_cores
    nw = num_cores * info.num_subcores
    bpw = n // nw
    mesh = plsc.VectorSubcoreMesh(core_axis_name="c", subcore_axis_name="s")

    @functools.partial(
        pl.kernel,
        mesh=mesh,
        out_type=jax.ShapeDtypeStruct((n, d), table.dtype),
        scratch_types=[
            pltpu.VMEM((bpw,), jnp.int32),
            pltpu.VMEM((bpw, d), table.dtype),
            pltpu.SemaphoreType.DMA,
        ],
    )
    def _gather(idx_hbm, table_hbm, out_hbm, idx_v, rows_v, sem):
        wid = lax.axis_index("s") * num_cores + lax.axis_index("c")
        base = wid * bpw
        pltpu.sync_copy(idx_hbm.at[pl.ds(base, bpw)], idx_v)
        pltpu.async_copy(table_hbm.at[idx_v], rows_v, sem).wait()
        pltpu.sync_copy(rows_v, out_hbm.at[pl.ds(base, bpw)])

    return _gather(idx_flat, table)


def kernel(x, emb_weight):
    flat = x.reshape(-1, _DIM)
    encoding_indices, loss_v = _dist_argmin(flat, emb_weight)
    quantized = _sc_gather(encoding_indices.reshape(-1), emb_weight)
    return quantized.reshape(x.shape), loss_v[0], encoding_indices


# speedup vs baseline: 6.1525x; 1.0139x over previous
import functools

import jax
import jax.numpy as jnp
from jax import lax
from jax.experimental import pallas as pl
from jax.experimental.pallas import tpu as pltpu
from jax.experimental.pallas import tpu_sc as plsc

_NUM_EMB = 8192
_DIM = 256
_COMMIT = 0.25
_BM = 512
_BN = 2048


def _vq_dist_argmin_body(total_elems, xs_ref, e_ref, x2_ref, e2_ref,
                         idx_ref, loss_ref, minval, minidx, lossval, acc):
    c = pl.program_id(0)
    r = pl.program_id(1)
    nc = pl.num_programs(0)
    nr = pl.num_programs(1)
    rows = pl.ds(r * _BM, _BM)

    @pl.when(jnp.logical_and(c == 0, r == 0))
    def _():
        acc[0] = jnp.float32(0.0)

    @pl.when(c == 0)
    def _():
        minval[rows, :] = jnp.full((_BM, 1), jnp.inf, jnp.float32)
        minidx[rows, :] = jnp.zeros((_BM, 1), jnp.int32)
        lossval[rows, :] = jnp.zeros((_BM, 1), jnp.float32)

    m = lax.dot_general(xs_ref[...], e_ref[...], (((1,), (1,)), ((), ())),
                        preferred_element_type=jnp.float32)
    d2 = (x2_ref[...] + e2_ref[...]) + m
    dist = jnp.sqrt(jnp.maximum(d2, 0.0))

    lmin = jnp.min(dist, axis=1, keepdims=True)
    cols = lax.broadcasted_iota(jnp.int32, dist.shape, 1)
    larg = jnp.min(jnp.where(dist == lmin, cols, _NUM_EMB), axis=1,
                   keepdims=True) + c * _BN

    prev = minval[rows, :]
    better = lmin < prev
    minidx[rows, :] = jnp.where(better, larg, minidx[rows, :])
    minval[rows, :] = jnp.where(better, lmin, prev).astype(jnp.bfloat16).astype(jnp.float32)
    lossval[rows, :] = jnp.where(better, lmin, lossval[rows, :])

    @pl.when(c == nc - 1)
    def _():
        idx_ref[...] = minidx[rows, :]
        lv = lossval[rows, :]
        acc[0] = acc[0] + jnp.sum(lv * lv)

        @pl.when(r == nr - 1)
        def _():
            loss_ref[0] = (acc[0] * jnp.float32(1.0 + _COMMIT)
                           / jnp.float32(total_elems))


def _dist_argmin(flat, emb_weight):
    n = flat.shape[0]
    grid = (_NUM_EMB // _BN, n // _BM)
    xs = (flat * -2.0).astype(jnp.bfloat16)
    ebf = emb_weight.astype(jnp.bfloat16)
    x2 = jnp.sum(flat * flat, axis=1)[:, None]
    e2 = jnp.sum(emb_weight * emb_weight, axis=1)[None, :]
    return pl.pallas_call(
        functools.partial(_vq_dist_argmin_body, n * _DIM),
        grid=grid,
        in_specs=[
            pl.BlockSpec((_BM, _DIM), lambda c, r: (r, 0)),
            pl.BlockSpec((_BN, _DIM), lambda c, r: (c, 0)),
            pl.BlockSpec((_BM, 1), lambda c, r: (r, 0)),
            pl.BlockSpec((1, _BN), lambda c, r: (0, c)),
        ],
        out_specs=[
            pl.BlockSpec((_BM, 1), lambda c, r: (r, 0)),
            pl.BlockSpec(memory_space=pltpu.SMEM),
        ],
        out_shape=[
            jax.ShapeDtypeStruct((n, 1), jnp.int32),
            jax.ShapeDtypeStruct((1,), jnp.float32),
        ],
        scratch_shapes=[
            pltpu.VMEM((n, 1), jnp.float32),
            pltpu.VMEM((n, 1), jnp.int32),
            pltpu.VMEM((n, 1), jnp.float32),
            pltpu.SMEM((1,), jnp.float32),
        ],
    )(xs, ebf, x2, e2)


def _sc_gather(idx_flat, table):
    n = idx_flat.shape[0]
    d = table.shape[1]
    info = plsc.get_sparse_core_info()
    num_cores = info.num_cores
    nw = num_cores * info.num_subcores
    bpw = n // nw
    mesh = plsc.VectorSubcoreMesh(core_axis_name="c", subcore_axis_name="s")

    @functools.partial(
        pl.kernel,
        mesh=mesh,
        out_type=jax.ShapeDtypeStruct((n, d), table.dtype),
        scratch_types=[
            pltpu.VMEM((bpw,), jnp.int32),
            pltpu.VMEM((bpw, d), table.dtype),
            pltpu.SemaphoreType.DMA,
        ],
    )
    def _gather(idx_hbm, table_hbm, out_hbm, idx_v, rows_v, sem):
        wid = lax.axis_index("s") * num_cores + lax.axis_index("c")
        base = wid * bpw
        pltpu.sync_copy(idx_hbm.at[pl.ds(base, bpw)], idx_v)
        pltpu.async_copy(table_hbm.at[idx_v], rows_v, sem).wait()
        pltpu.sync_copy(rows_v, out_hbm.at[pl.ds(base, bpw)])

    return _gather(idx_flat, table)


def kernel(x, emb_weight):
    flat = x.reshape(-1, _DIM)
    encoding_indices, loss_v = _dist_argmin(flat, emb_weight)
    quantized = _sc_gather(encoding_indices.reshape(-1), emb_weight)
    return quantized.reshape(x.shape), loss_v[0], encoding_indices


# speedup vs baseline: 6.6227x; 1.0764x over previous
import functools

import jax
import jax.numpy as jnp
from jax import lax
from jax.experimental import pallas as pl
from jax.experimental.pallas import tpu as pltpu
from jax.experimental.pallas import tpu_sc as plsc

_NUM_EMB = 8192
_DIM = 256
_COMMIT = 0.25
_BM = 512
_BN = 2048
_W = 128


def _vq_dist_argmin_body(total_elems, xs_ref, e_ref, x2_ref, e2_ref,
                         idx_ref, loss_ref, minval, minidx, lossval, acc):
    c = pl.program_id(0)
    r = pl.program_id(1)
    nc = pl.num_programs(0)
    nr = pl.num_programs(1)
    rows = pl.ds(r * _BM, _BM)

    @pl.when(jnp.logical_and(c == 0, r == 0))
    def _():
        acc[0] = jnp.float32(0.0)

    @pl.when(c == 0)
    def _():
        minval[rows, :] = jnp.full((_BM, 1), jnp.inf, jnp.float32)
        minidx[rows, :] = jnp.zeros((_BM, 1), jnp.int32)
        lossval[rows, :] = jnp.zeros((_BM, 1), jnp.float32)

    m = lax.dot_general(xs_ref[...], e_ref[...], (((1,), (1,)), ((), ())),
                        preferred_element_type=jnp.float32)
    x2 = x2_ref[...]

    run_v = None
    for k in range(_BN // _W):
        cs = slice(k * _W, (k + 1) * _W)
        d2 = (x2 + e2_ref[:, cs]) + m[:, cs]
        dv = jnp.sqrt(jnp.maximum(d2, 0.0))
        if run_v is None:
            run_v = dv
            run_k = jnp.zeros(dv.shape, jnp.int32)
        else:
            sel = dv < run_v
            run_v = jnp.minimum(run_v, dv)
            run_k = jnp.where(sel, k, run_k)

    lmin = jnp.min(run_v, axis=1, keepdims=True)
    lanes = lax.broadcasted_iota(jnp.int32, run_v.shape, 1)
    colfull = run_k * _W + lanes
    larg = jnp.min(jnp.where(run_v == lmin, colfull, _NUM_EMB), axis=1,
                   keepdims=True) + c * _BN

    prev = minval[rows, :]
    better = lmin < prev
    minidx[rows, :] = jnp.where(better, larg, minidx[rows, :])
    minval[rows, :] = jnp.where(better, lmin, prev).astype(jnp.bfloat16).astype(jnp.float32)
    lossval[rows, :] = jnp.where(better, lmin, lossval[rows, :])

    @pl.when(c == nc - 1)
    def _():
        idx_ref[...] = minidx[rows, :]
        lv = lossval[rows, :]
        acc[0] = acc[0] + jnp.sum(lv * lv)

        @pl.when(r == nr - 1)
        def _():
            loss_ref[0] = (acc[0] * jnp.float32(1.0 + _COMMIT)
                           / jnp.float32(total_elems))


def _dist_argmin(flat, emb_weight):
    n = flat.shape[0]
    grid = (_NUM_EMB // _BN, n // _BM)
    xs = (flat * -2.0).astype(jnp.bfloat16)
    ebf = emb_weight.astype(jnp.bfloat16)
    x2 = jnp.sum(flat * flat, axis=1)[:, None]
    e2 = jnp.sum(emb_weight * emb_weight, axis=1)[None, :]
    return pl.pallas_call(
        functools.partial(_vq_dist_argmin_body, n * _DIM),
        grid=grid,
        in_specs=[
            pl.BlockSpec((_BM, _DIM), lambda c, r: (r, 0)),
            pl.BlockSpec((_BN, _DIM), lambda c, r: (c, 0)),
            pl.BlockSpec((_BM, 1), lambda c, r: (r, 0)),
            pl.BlockSpec((1, _BN), lambda c, r: (0, c)),
        ],
        out_specs=[
            pl.BlockSpec((_BM, 1), lambda c, r: (r, 0)),
            pl.BlockSpec(memory_space=pltpu.SMEM),
        ],
        out_shape=[
            jax.ShapeDtypeStruct((n, 1), jnp.int32),
            jax.ShapeDtypeStruct((1,), jnp.float32),
        ],
        scratch_shapes=[
            pltpu.VMEM((n, 1), jnp.float32),
            pltpu.VMEM((n, 1), jnp.int32),
            pltpu.VMEM((n, 1), jnp.float32),
            pltpu.SMEM((1,), jnp.float32),
        ],
    )(xs, ebf, x2, e2)


def _sc_gather(idx_flat, table):
    n = idx_flat.shape[0]
    d = table.shape[1]
    info = plsc.get_sparse_core_info()
    num_cores = info.num_cores
    nw = num_cores * info.num_subcores
    bpw = n // nw
    mesh = plsc.VectorSubcoreMesh(core_axis_name="c", subcore_axis_name="s")

    @functools.partial(
        pl.kernel,
        mesh=mesh,
        out_type=jax.ShapeDtypeStruct((n, d), table.dtype),
        scratch_types=[
            pltpu.VMEM((bpw,), jnp.int32),
            pltpu.VMEM((bpw, d), table.dtype),
            pltpu.SemaphoreType.DMA,
        ],
    )
    def _gather(idx_hbm, table_hbm, out_hbm, idx_v, rows_v, sem):
        wid = lax.axis_index("s") * num_cores + lax.axis_index("c")
        base = wid * bpw
        pltpu.sync_copy(idx_hbm.at[pl.ds(base, bpw)], idx_v)
        pltpu.async_copy(table_hbm.at[idx_v], rows_v, sem).wait()
        pltpu.sync_copy(rows_v, out_hbm.at[pl.ds(base, bpw)])

    return _gather(idx_flat, table)


def kernel(x, emb_weight):
    flat = x.reshape(-1, _DIM)
    encoding_indices, loss_v = _dist_argmin(flat, emb_weight)
    quantized = _sc_gather(encoding_indices.reshape(-1), emb_weight)
    return quantized.reshape(x.shape), loss_v[0], encoding_indices


# speedup vs baseline: 6.9599x; 1.0509x over previous
import functools

import jax
import jax.numpy as jnp
from jax import lax
from jax.experimental import pallas as pl
from jax.experimental.pallas import tpu as pltpu
from jax.experimental.pallas import tpu_sc as plsc

_NUM_EMB = 8192
_DIM = 256
_COMMIT = 0.25
_BM = 512
_BN = 2048
_W = 256


def _vq_dist_argmin_body(total_elems, xs_ref, e_ref, x2_ref, e2_ref,
                         idx_ref, loss_ref, minval, minidx, lossval, acc):
    c = pl.program_id(0)
    r = pl.program_id(1)
    nc = pl.num_programs(0)
    nr = pl.num_programs(1)
    rows = pl.ds(r * _BM, _BM)

    @pl.when(jnp.logical_and(c == 0, r == 0))
    def _():
        acc[0] = jnp.float32(0.0)

    @pl.when(c == 0)
    def _():
        minval[rows, :] = jnp.full((_BM, 1), jnp.inf, jnp.float32)
        minidx[rows, :] = jnp.zeros((_BM, 1), jnp.int32)
        lossval[rows, :] = jnp.zeros((_BM, 1), jnp.float32)

    xsb = xs_ref[...]
    x2 = x2_ref[...]

    run_v = None
    for k in range(_BN // _W):
        cs = slice(k * _W, (k + 1) * _W)
        m = lax.dot_general(xsb, e_ref[cs, :], (((1,), (1,)), ((), ())),
                            preferred_element_type=jnp.float32)
        d2 = (x2 + e2_ref[:, cs]) + m
        dv = jnp.sqrt(jnp.maximum(d2, 0.0))
        if run_v is None:
            run_v = dv
            run_k = jnp.zeros(dv.shape, jnp.int32)
        else:
            sel = dv < run_v
            run_v = jnp.where(sel, dv, run_v)
            run_k = jnp.where(sel, k, run_k)

    lmin = jnp.min(run_v, axis=1, keepdims=True)
    lanes = lax.broadcasted_iota(jnp.int32, run_v.shape, 1)
    colfull = run_k * _W + lanes
    larg = jnp.min(jnp.where(run_v == lmin, colfull, _NUM_EMB), axis=1,
                   keepdims=True) + c * _BN

    prev = minval[rows, :]
    better = lmin < prev
    minidx[rows, :] = jnp.where(better, larg, minidx[rows, :])
    minval[rows, :] = jnp.where(better, lmin, prev).astype(jnp.bfloat16).astype(jnp.float32)
    lossval[rows, :] = jnp.where(better, lmin, lossval[rows, :])

    @pl.when(c == nc - 1)
    def _():
        idx_ref[...] = minidx[rows, :]
        lv = lossval[rows, :]
        acc[0] = acc[0] + jnp.sum(lv * lv)

        @pl.when(r == nr - 1)
        def _():
            loss_ref[0] = (acc[0] * jnp.float32(1.0 + _COMMIT)
                           / jnp.float32(total_elems))


def _dist_argmin(flat, emb_weight):
    n = flat.shape[0]
    grid = (_NUM_EMB // _BN, n // _BM)
    xs = (flat * -2.0).astype(jnp.bfloat16)
    ebf = emb_weight.astype(jnp.bfloat16)
    x2 = jnp.sum(flat * flat, axis=1)[:, None]
    e2 = jnp.sum(emb_weight * emb_weight, axis=1)[None, :]
    return pl.pallas_call(
        functools.partial(_vq_dist_argmin_body, n * _DIM),
        grid=grid,
        in_specs=[
            pl.BlockSpec((_BM, _DIM), lambda c, r: (r, 0)),
            pl.BlockSpec((_BN, _DIM), lambda c, r: (c, 0)),
            pl.BlockSpec((_BM, 1), lambda c, r: (r, 0)),
            pl.BlockSpec((1, _BN), lambda c, r: (0, c)),
        ],
        out_specs=[
            pl.BlockSpec((_BM, 1), lambda c, r: (r, 0)),
            pl.BlockSpec(memory_space=pltpu.SMEM),
        ],
        out_shape=[
            jax.ShapeDtypeStruct((n, 1), jnp.int32),
            jax.ShapeDtypeStruct((1,), jnp.float32),
        ],
        scratch_shapes=[
            pltpu.VMEM((n, 1), jnp.float32),
            pltpu.VMEM((n, 1), jnp.int32),
            pltpu.VMEM((n, 1), jnp.float32),
            pltpu.SMEM((1,), jnp.float32),
        ],
    )(xs, ebf, x2, e2)


def _sc_gather(idx_flat, table):
    n = idx_flat.shape[0]
    d = table.shape[1]
    info = plsc.get_sparse_core_info()
    num_cores = info.num_cores
    nw = num_cores * info.num_subcores
    bpw = n // nw
    mesh = plsc.VectorSubcoreMesh(core_axis_name="c", subcore_axis_name="s")

    @functools.partial(
        pl.kernel,
        mesh=mesh,
        out_type=jax.ShapeDtypeStruct((n, d), table.dtype),
        scratch_types=[
            pltpu.VMEM((bpw,), jnp.int32),
            pltpu.VMEM((bpw, d), table.dtype),
            pltpu.SemaphoreType.DMA,
        ],
    )
    def _gather(idx_hbm, table_hbm, out_hbm, idx_v, rows_v, sem):
        wid = lax.axis_index("s") * num_cores + lax.axis_index("c")
        base = wid * bpw
        pltpu.sync_copy(idx_hbm.at[pl.ds(base, bpw)], idx_v)
        pltpu.async_copy(table_hbm.at[idx_v], rows_v, sem).wait()
        pltpu.sync_copy(rows_v, out_hbm.at[pl.ds(base, bpw)])

    return _gather(idx_flat, table)


def kernel(x, emb_weight):
    flat = x.reshape(-1, _DIM)
    encoding_indices, loss_v = _dist_argmin(flat, emb_weight)
    quantized = _sc_gather(encoding_indices.reshape(-1), emb_weight)
    return quantized.reshape(x.shape), loss_v[0], encoding_indices


# speedup vs baseline: 7.7773x; 1.1174x over previous
import functools

import jax
import jax.numpy as jnp
from jax import lax
from jax.experimental import pallas as pl
from jax.experimental.pallas import tpu as pltpu
from jax.experimental.pallas import tpu_sc as plsc

_NUM_EMB = 8192
_DIM = 256
_COMMIT = 0.25
_BM = 512
_BN = 2048
_W = 256


def _vq_dist_argmin_body(total_elems, xs_ref, e_ref, x2_ref, e2_ref, colf_ref,
                         idx_ref, loss_ref, d2buf, minval, minidx, lossval, acc):
    c = pl.program_id(0)
    r = pl.program_id(1)
    nc = pl.num_programs(0)
    nr = pl.num_programs(1)
    rows = pl.ds(r * _BM, _BM)

    @pl.when(jnp.logical_and(c == 0, r == 0))
    def _():
        acc[0] = jnp.float32(0.0)

    @pl.when(c == 0)
    def _():
        minval[rows, :] = jnp.full((_BM, 1), jnp.inf, jnp.float32)
        minidx[rows, :] = jnp.zeros((_BM, 1), jnp.int32)
        lossval[rows, :] = jnp.zeros((_BM, 1), jnp.float32)

    xsb = xs_ref[...]
    x2 = x2_ref[...]

    for k in range(_BN // _W):
        cs = slice(k * _W, (k + 1) * _W)
        m = lax.dot_general(xsb, e_ref[cs, :], (((1,), (1,)), ((), ())),
                            preferred_element_type=jnp.float32)
        d2buf[:, cs] = (x2 + e2_ref[:, cs]) + m

    v = d2buf[...]
    lmin = jnp.min(v, axis=1, keepdims=True)
    d2c = jnp.maximum(lmin, 0.0)
    s = jnp.sqrt(d2c)

    c0 = s * s
    c0i = lax.bitcast_convert_type(c0, jnp.int32)
    hi = c0
    for k in (0, 1, 2, 3):
        y = lax.bitcast_convert_type(c0i + k, jnp.float32)
        ok = jnp.sqrt(y) == s
        hi = jnp.where(ok, y, hi)
    hi = jnp.maximum(hi, lmin)

    cand = jnp.where(v <= hi, colf_ref[...], jnp.float32(3.0e38))
    larg = jnp.min(cand, axis=1, keepdims=True).astype(jnp.int32)

    prev = minval[rows, :]
    better = s < prev
    minidx[rows, :] = jnp.where(better, larg, minidx[rows, :])
    minval[rows, :] = jnp.where(better, s, prev).astype(jnp.bfloat16).astype(jnp.float32)
    lossval[rows, :] = jnp.where(better, d2c, lossval[rows, :])

    @pl.when(c == nc - 1)
    def _():
        idx_ref[...] = minidx[rows, :]
        acc[0] = acc[0] + jnp.sum(lossval[rows, :])

        @pl.when(r == nr - 1)
        def _():
            loss_ref[0] = (acc[0] * jnp.float32(1.0 + _COMMIT)
                           / jnp.float32(total_elems))


def _dist_argmin(flat, emb_weight):
    n = flat.shape[0]
    ne = emb_weight.shape[0]
    grid = (ne // _BN, n // _BM)
    xs = (flat * -2.0).astype(jnp.bfloat16)
    ebf = emb_weight.astype(jnp.bfloat16)
    x2 = jnp.sum(flat * flat, axis=1)[:, None]
    e2 = jnp.sum(emb_weight * emb_weight, axis=1)[None, :]
    colf = jnp.arange(ne, dtype=jnp.float32)[None, :]
    return pl.pallas_call(
        functools.partial(_vq_dist_argmin_body, n * _DIM),
        grid=grid,
        in_specs=[
            pl.BlockSpec((_BM, _DIM), lambda c, r: (r, 0)),
            pl.BlockSpec((_BN, _DIM), lambda c, r: (c, 0)),
            pl.BlockSpec((_BM, 1), lambda c, r: (r, 0)),
            pl.BlockSpec((1, _BN), lambda c, r: (0, c)),
            pl.BlockSpec((1, _BN), lambda c, r: (0, c)),
        ],
        out_specs=[
            pl.BlockSpec((_BM, 1), lambda c, r: (r, 0)),
            pl.BlockSpec(memory_space=pltpu.SMEM),
        ],
        out_shape=[
            jax.ShapeDtypeStruct((n, 1), jnp.int32),
            jax.ShapeDtypeStruct((1,), jnp.float32),
        ],
        scratch_shapes=[
            pltpu.VMEM((_BM, _BN), jnp.float32),
            pltpu.VMEM((n, 1), jnp.float32),
            pltpu.VMEM((n, 1), jnp.int32),
            pltpu.VMEM((n, 1), jnp.float32),
            pltpu.SMEM((1,), jnp.float32),
        ],
    )(xs, ebf, x2, e2, colf)


def _sc_gather(idx_flat, table):
    n = idx_flat.shape[0]
    d = table.shape[1]
    info = plsc.get_sparse_core_info()
    num_cores = info.num_cores
    nw = num_cores * info.num_subcores
    bpw = n // nw
    mesh = plsc.VectorSubcoreMesh(core_axis_name="c", subcore_axis_name="s")

    @functools.partial(
        pl.kernel,
        mesh=mesh,
        out_type=jax.ShapeDtypeStruct((n, d), table.dtype),
        scratch_types=[
            pltpu.VMEM((bpw,), jnp.int32),
            pltpu.VMEM((bpw, d), table.dtype),
            pltpu.SemaphoreType.DMA,
        ],
    )
    def _gather(idx_hbm, table_hbm, out_hbm, idx_v, rows_v, sem):
        wid = lax.axis_index("s") * num_cores + lax.axis_index("c")
        base = wid * bpw
        pltpu.sync_copy(idx_hbm.at[pl.ds(base, bpw)], idx_v)
        pltpu.async_copy(table_hbm.at[idx_v], rows_v, sem).wait()
        pltpu.sync_copy(rows_v, out_hbm.at[pl.ds(base, bpw)])

    return _gather(idx_flat, table)


def kernel(x, emb_weight):
    flat = x.reshape(-1, _DIM)
    encoding_indices, loss_v = _dist_argmin(flat, emb_weight)
    quantized = _sc_gather(encoding_indices.reshape(-1), emb_weight)
    return quantized.reshape(x.shape), loss_v[0], encoding_indices


# speedup vs baseline: 8.4404x; 1.0853x over previous
import functools

import jax
import jax.numpy as jnp
from jax import lax
from jax.experimental import pallas as pl
from jax.experimental.pallas import tpu as pltpu
from jax.experimental.pallas import tpu_sc as plsc

_NUM_EMB = 8192
_DIM = 256
_COMMIT = 0.25
_BM = 1024
_BN = 2048
_W = 256


def _vq_dist_argmin_body(total_elems, xs_ref, e_ref, x2_ref, e2_ref, colf_ref,
                         idx_ref, loss_ref, d2buf, minval, minidx, lossval, acc):
    c = pl.program_id(0)
    r = pl.program_id(1)
    nc = pl.num_programs(0)
    nr = pl.num_programs(1)
    rows = pl.ds(r * _BM, _BM)

    @pl.when(jnp.logical_and(c == 0, r == 0))
    def _():
        acc[0] = jnp.float32(0.0)

    @pl.when(c == 0)
    def _():
        minval[rows, :] = jnp.full((_BM, 1), jnp.inf, jnp.float32)
        minidx[rows, :] = jnp.zeros((_BM, 1), jnp.int32)
        lossval[rows, :] = jnp.zeros((_BM, 1), jnp.float32)

    xsb = xs_ref[...]
    x2 = x2_ref[...]

    for k in range(_BN // _W):
        cs = slice(k * _W, (k + 1) * _W)
        m = lax.dot_general(xsb, e_ref[cs, :], (((1,), (1,)), ((), ())),
                            preferred_element_type=jnp.float32)
        d2buf[:, cs] = (x2 + e2_ref[:, cs]) + m

    v = d2buf[...]
    lmin = jnp.min(v, axis=1, keepdims=True)
    d2c = jnp.maximum(lmin, 0.0)
    s = jnp.sqrt(d2c)

    c0 = s * s
    c0i = lax.bitcast_convert_type(c0, jnp.int32)
    hi = c0
    for k in (0, 1, 2, 3):
        y = lax.bitcast_convert_type(c0i + k, jnp.float32)
        ok = jnp.sqrt(y) == s
        hi = jnp.where(ok, y, hi)
    hi = jnp.maximum(hi, lmin)

    cand = jnp.where(v <= hi, colf_ref[...], jnp.float32(3.0e38))
    larg = jnp.min(cand, axis=1, keepdims=True).astype(jnp.int32)

    prev = minval[rows, :]
    better = s < prev
    minidx[rows, :] = jnp.where(better, larg, minidx[rows, :])
    minval[rows, :] = jnp.where(better, s, prev).astype(jnp.bfloat16).astype(jnp.float32)
    lossval[rows, :] = jnp.where(better, d2c, lossval[rows, :])

    @pl.when(c == nc - 1)
    def _():
        idx_ref[...] = minidx[rows, :]
        acc[0] = acc[0] + jnp.sum(lossval[rows, :])

        @pl.when(r == nr - 1)
        def _():
            loss_ref[0] = (acc[0] * jnp.float32(1.0 + _COMMIT)
                           / jnp.float32(total_elems))


def _dist_argmin(flat, emb_weight):
    n = flat.shape[0]
    ne = emb_weight.shape[0]
    grid = (ne // _BN, n // _BM)
    xs = (flat * -2.0).astype(jnp.bfloat16)
    ebf = emb_weight.astype(jnp.bfloat16)
    x2 = jnp.sum(flat * flat, axis=1)[:, None]
    e2 = jnp.sum(emb_weight * emb_weight, axis=1)[None, :]
    colf = jnp.arange(ne, dtype=jnp.float32)[None, :]
    return pl.pallas_call(
        functools.partial(_vq_dist_argmin_body, n * _DIM),
        grid=grid,
        in_specs=[
            pl.BlockSpec((_BM, _DIM), lambda c, r: (r, 0)),
            pl.BlockSpec((_BN, _DIM), lambda c, r: (c, 0)),
            pl.BlockSpec((_BM, 1), lambda c, r: (r, 0)),
            pl.BlockSpec((1, _BN), lambda c, r: (0, c)),
            pl.BlockSpec((1, _BN), lambda c, r: (0, c)),
        ],
        out_specs=[
            pl.BlockSpec((_BM, 1), lambda c, r: (r, 0)),
            pl.BlockSpec(memory_space=pltpu.SMEM),
        ],
        out_shape=[
            jax.ShapeDtypeStruct((n, 1), jnp.int32),
            jax.ShapeDtypeStruct((1,), jnp.float32),
        ],
        scratch_shapes=[
            pltpu.VMEM((_BM, _BN), jnp.float32),
            pltpu.VMEM((n, 1), jnp.float32),
            pltpu.VMEM((n, 1), jnp.int32),
            pltpu.VMEM((n, 1), jnp.float32),
            pltpu.SMEM((1,), jnp.float32),
        ],
    )(xs, ebf, x2, e2, colf)


def _sc_gather(idx_flat, table):
    n = idx_flat.shape[0]
    d = table.shape[1]
    info = plsc.get_sparse_core_info()
    num_cores = info.num_cores
    nw = num_cores * info.num_subcores
    bpw = n // nw
    mesh = plsc.VectorSubcoreMesh(core_axis_name="c", subcore_axis_name="s")

    @functools.partial(
        pl.kernel,
        mesh=mesh,
        out_type=jax.ShapeDtypeStruct((n, d), table.dtype),
        scratch_types=[
            pltpu.VMEM((bpw,), jnp.int32),
            pltpu.VMEM((bpw, d), table.dtype),
            pltpu.SemaphoreType.DMA,
        ],
    )
    def _gather(idx_hbm, table_hbm, out_hbm, idx_v, rows_v, sem):
        wid = lax.axis_index("s") * num_cores + lax.axis_index("c")
        base = wid * bpw
        pltpu.sync_copy(idx_hbm.at[pl.ds(base, bpw)], idx_v)
        pltpu.async_copy(table_hbm.at[idx_v], rows_v, sem).wait()
        pltpu.sync_copy(rows_v, out_hbm.at[pl.ds(base, bpw)])

    return _gather(idx_flat, table)


def kernel(x, emb_weight):
    flat = x.reshape(-1, _DIM)
    encoding_indices, loss_v = _dist_argmin(flat, emb_weight)
    quantized = _sc_gather(encoding_indices.reshape(-1), emb_weight)
    return quantized.reshape(x.shape), loss_v[0], encoding_indices


# speedup vs baseline: 8.8241x; 1.0455x over previous
import functools

import jax
import jax.numpy as jnp
from jax import lax
from jax.experimental import pallas as pl
from jax.experimental.pallas import tpu as pltpu
from jax.experimental.pallas import tpu_sc as plsc

_NUM_EMB = 8192
_DIM = 256
_COMMIT = 0.25
_BM = 2048
_BN = 2048
_W = 256


def _vq_dist_argmin_body(total_elems, xs_ref, e_ref, x2_ref, e2_ref, colf_ref,
                         idx_ref, loss_ref, d2buf, minval, minidx, lossval, acc):
    c = pl.program_id(0)
    r = pl.program_id(1)
    nc = pl.num_programs(0)
    nr = pl.num_programs(1)
    rows = pl.ds(r * _BM, _BM)

    @pl.when(jnp.logical_and(c == 0, r == 0))
    def _():
        acc[0] = jnp.float32(0.0)

    @pl.when(c == 0)
    def _():
        minval[rows, :] = jnp.full((_BM, 1), jnp.inf, jnp.float32)
        minidx[rows, :] = jnp.zeros((_BM, 1), jnp.int32)
        lossval[rows, :] = jnp.zeros((_BM, 1), jnp.float32)

    xsb = xs_ref[...]
    x2 = x2_ref[...]

    for k in range(_BN // _W):
        cs = slice(k * _W, (k + 1) * _W)
        m = lax.dot_general(xsb, e_ref[cs, :], (((1,), (1,)), ((), ())),
                            preferred_element_type=jnp.float32)
        d2buf[:, cs] = (x2 + e2_ref[:, cs]) + m

    v = d2buf[...]
    lmin = jnp.min(v, axis=1, keepdims=True)
    d2c = jnp.maximum(lmin, 0.0)
    s = jnp.sqrt(d2c)

    c0 = s * s
    c0i = lax.bitcast_convert_type(c0, jnp.int32)
    hi = c0
    for k in (0, 1, 2, 3):
        y = lax.bitcast_convert_type(c0i + k, jnp.float32)
        ok = jnp.sqrt(y) == s
        hi = jnp.where(ok, y, hi)
    hi = jnp.maximum(hi, lmin)

    cand = jnp.where(v <= hi, colf_ref[...], jnp.float32(3.0e38))
    larg = jnp.min(cand, axis=1, keepdims=True).astype(jnp.int32)

    prev = minval[rows, :]
    better = s < prev
    minidx[rows, :] = jnp.where(better, larg, minidx[rows, :])
    minval[rows, :] = jnp.where(better, s, prev).astype(jnp.bfloat16).astype(jnp.float32)
    lossval[rows, :] = jnp.where(better, d2c, lossval[rows, :])

    @pl.when(c == nc - 1)
    def _():
        idx_ref[...] = minidx[rows, :]
        acc[0] = acc[0] + jnp.sum(lossval[rows, :])

        @pl.when(r == nr - 1)
        def _():
            loss_ref[0] = (acc[0] * jnp.float32(1.0 + _COMMIT)
                           / jnp.float32(total_elems))


def _dist_argmin(flat, emb_weight):
    n = flat.shape[0]
    ne = emb_weight.shape[0]
    grid = (ne // _BN, n // _BM)
    xs = (flat * -2.0).astype(jnp.bfloat16)
    ebf = emb_weight.astype(jnp.bfloat16)
    x2 = jnp.sum(flat * flat, axis=1)[:, None]
    e2 = jnp.sum(emb_weight * emb_weight, axis=1)[None, :]
    colf = jnp.arange(ne, dtype=jnp.float32)[None, :]
    return pl.pallas_call(
        functools.partial(_vq_dist_argmin_body, n * _DIM),
        grid=grid,
        in_specs=[
            pl.BlockSpec((_BM, _DIM), lambda c, r: (r, 0)),
            pl.BlockSpec((_BN, _DIM), lambda c, r: (c, 0)),
            pl.BlockSpec((_BM, 1), lambda c, r: (r, 0)),
            pl.BlockSpec((1, _BN), lambda c, r: (0, c)),
            pl.BlockSpec((1, _BN), lambda c, r: (0, c)),
        ],
        out_specs=[
            pl.BlockSpec((_BM, 1), lambda c, r: (r, 0)),
            pl.BlockSpec(memory_space=pltpu.SMEM),
        ],
        out_shape=[
            jax.ShapeDtypeStruct((n, 1), jnp.int32),
            jax.ShapeDtypeStruct((1,), jnp.float32),
        ],
        scratch_shapes=[
            pltpu.VMEM((_BM, _BN), jnp.float32),
            pltpu.VMEM((n, 1), jnp.float32),
            pltpu.VMEM((n, 1), jnp.int32),
            pltpu.VMEM((n, 1), jnp.float32),
            pltpu.SMEM((1,), jnp.float32),
        ],
    )(xs, ebf, x2, e2, colf)


def _sc_gather(idx_flat, table):
    n = idx_flat.shape[0]
    d = table.shape[1]
    info = plsc.get_sparse_core_info()
    num_cores = info.num_cores
    nw = num_cores * info.num_subcores
    bpw = n // nw
    mesh = plsc.VectorSubcoreMesh(core_axis_name="c", subcore_axis_name="s")

    @functools.partial(
        pl.kernel,
        mesh=mesh,
        out_type=jax.ShapeDtypeStruct((n, d), table.dtype),
        scratch_types=[
            pltpu.VMEM((bpw,), jnp.int32),
            pltpu.VMEM((bpw, d), table.dtype),
            pltpu.SemaphoreType.DMA,
        ],
    )
    def _gather(idx_hbm, table_hbm, out_hbm, idx_v, rows_v, sem):
        wid = lax.axis_index("s") * num_cores + lax.axis_index("c")
        base = wid * bpw
        pltpu.sync_copy(idx_hbm.at[pl.ds(base, bpw)], idx_v)
        pltpu.async_copy(table_hbm.at[idx_v], rows_v, sem).wait()
        pltpu.sync_copy(rows_v, out_hbm.at[pl.ds(base, bpw)])

    return _gather(idx_flat, table)


def kernel(x, emb_weight):
    flat = x.reshape(-1, _DIM)
    encoding_indices, loss_v = _dist_argmin(flat, emb_weight)
    quantized = _sc_gather(encoding_indices.reshape(-1), emb_weight)
    return quantized.reshape(x.shape), loss_v[0], encoding_indices
